# baseline (speedup 1.0000x reference)
import jax
import jax.numpy as jnp
import numpy as np
from jax.experimental import pallas as pl
from jax.experimental.pallas import tpu as pltpu
from jax.sharding import Mesh, PartitionSpec as P

_N, _C, _H, _W = 8, 32, 128, 128
_Q = 16


def _minmax_body(x_ref, o_ref):
    x = x_ref[...]
    o_ref[0] = jnp.min(x)
    o_ref[1] = jnp.max(x)


def _shift_w(y, d, lane, mask):
    if d == 0:
        return y
    r = pltpu.roll(y, (-d) % _W, axis=2)
    if not mask:
        return r
    edge = _W - 1 if d == 1 else 0
    return jnp.where(lane == edge, 0, r)


def _shift_h(y, d, sub, mask):
    if d == 0:
        return y
    r = pltpu.roll(y, (-d) % _H, axis=1)
    if not mask:
        return r
    edge = _H - 1 if d == 1 else 0
    return jnp.where(sub == edge, 0, r)


def _shift_c(y, d):
    if d == 0:
        return y
    z = jnp.zeros((1, _H, _W), y.dtype)
    if d == 1:
        return jnp.concatenate([y[1:], z], axis=0)
    return jnp.concatenate([z, y[:-1]], axis=0)


def _main_body(mm_ref, f_ref, tab_ref, x_ref, o_ref):
    x = x_ref[0]
    xmin = mm_ref[0]
    xmax = mm_ref[1]
    q = jnp.float32(_Q)
    t = (x - xmin) / xmax * q
    idx = jnp.clip(jnp.floor(t).astype(jnp.int32), 0, _Q - 1)
    a8 = idx * (_Q // 2)

    lane = jax.lax.broadcasted_iota(jnp.int32, (_C, _H, _W), 2)
    sub = jax.lax.broadcasted_iota(jnp.int32, (_C, _H, _W), 1)

    tab = jnp.broadcast_to(tab_ref[0][None, None, :], (_C, _H, _W))

    acc = jnp.zeros((_C, _H, _W), jnp.float32)
    for dw in (-1, 0, 1):
        xw = _shift_w(x, dw, lane, True)
        bw = _shift_w(idx, dw, lane, True)
        for dh in (-1, 0, 1):
            xwh = _shift_h(xw, dh, sub, True)
            bwh = _shift_h(bw, dh, sub, True)
            for dc in (-1, 0, 1):
                xs = _shift_c(xwh, dc)
                bs = _shift_c(bwh, dc)
                pair = a8 + (bs >> 1)
                u = jnp.take_along_axis(tab, pair, axis=2)
                odd = (bs & 1) == 1
                bits = jnp.where(odd, u & jnp.int32(-65536), u << 16)
                val = pltpu.bitcast(bits, jnp.float32)
                ft = f_ref[(dc + 1) * 9 + (dh + 1) * 3 + (dw + 1)]
                acc = acc + (ft * xs) * val
    o_ref[0] = acc


def _pack_co_table(co_matrix):
    cb = co_matrix.reshape(-1).astype(jnp.bfloat16)
    u16 = jax.lax.bitcast_convert_type(cb, jnp.uint16).astype(jnp.uint32)
    packed = u16[0::2] | (u16[1::2] << 16)
    return packed.astype(jnp.int32).reshape(1, 128)


def _local_minmax(x_s):
    xr = x_s.reshape(x_s.shape[0] * _C * _H, _W)
    return pl.pallas_call(
        _minmax_body,
        out_shape=jax.ShapeDtypeStruct((2,), jnp.float32),
        in_specs=[pl.BlockSpec(memory_space=pltpu.VMEM)],
        out_specs=pl.BlockSpec(memory_space=pltpu.SMEM),
    )(xr)


def _main_call(mm, x_s, co_matrix, spatial_filter):
    tab = _pack_co_table(co_matrix)
    f = spatial_filter.reshape(27)
    nloc = x_s.shape[0]
    return pl.pallas_call(
        _main_body,
        grid=(nloc,),
        out_shape=jax.ShapeDtypeStruct((nloc, _C, _H, _W), jnp.float32),
        in_specs=[
            pl.BlockSpec(memory_space=pltpu.SMEM),
            pl.BlockSpec(memory_space=pltpu.SMEM),
            pl.BlockSpec((1, 128), lambda n: (0, 0)),
            pl.BlockSpec((1, _C, _H, _W), lambda n: (n, 0, 0, 0)),
        ],
        out_specs=pl.BlockSpec((1, _C, _H, _W), lambda n: (n, 0, 0, 0)),
        compiler_params=pltpu.CompilerParams(
            dimension_semantics=("arbitrary",),
            flags={"XLA_TPU_STORE_TO_LOAD_FORWARDING_WINDOW": 12288},
        ),
    )(mm, f, tab, x_s)


def kernel(x, co_matrix, spatial_filter):
    devs = jax.devices()
    if len(devs) < 2:
        mm = _local_minmax(x)
        return _main_call(mm, x, co_matrix, spatial_filter)

    mesh = Mesh(np.array(devs[:2]), ("d",))

    def shard_fn(x_full, co_s, f_s):
        mm = _local_minmax(x_full)
        i = jax.lax.axis_index("d")
        x_s = jax.lax.dynamic_slice_in_dim(x_full, i * (_N // 2), _N // 2, 0)
        return _main_call(mm, x_s, co_s, f_s)

    fn = jax.shard_map(
        shard_fn,
        mesh=mesh,
        in_specs=(P(), P(), P()),
        out_specs=P("d"),
        check_vma=False,
    )
    return fn(x, co_matrix, spatial_filter)


# speedup vs baseline: 1.0686x; 1.0686x over previous
import jax
import jax.numpy as jnp
import numpy as np
from jax.experimental import pallas as pl
from jax.experimental.pallas import tpu as pltpu
from jax.sharding import Mesh, PartitionSpec as P

_N, _C, _H, _W = 8, 32, 128, 128
_Q = 16


def _minmax_body(x_ref, o_ref):
    x = x_ref[...]
    o_ref[0] = jnp.min(x)
    o_ref[1] = jnp.max(x)


def _shift_w(y, d, lane, mask):
    if d == 0:
        return y
    r = pltpu.roll(y, (-d) % _W, axis=2)
    if not mask:
        return r
    edge = _W - 1 if d == 1 else 0
    return jnp.where(lane == edge, 0, r)


def _shift_h(y, d, sub, mask):
    if d == 0:
        return y
    r = pltpu.roll(y, (-d) % _H, axis=1)
    if not mask:
        return r
    edge = _H - 1 if d == 1 else 0
    return jnp.where(sub == edge, 0, r)


def _shift_c(y, d):
    if d == 0:
        return y
    z = jnp.zeros((1, _H, _W), y.dtype)
    if d == 1:
        return jnp.concatenate([y[1:], z], axis=0)
    return jnp.concatenate([z, y[:-1]], axis=0)


def _main_body(mm_ref, f_ref, tab_ref, x_ref, o_ref):
    x = x_ref[0]
    xmin = mm_ref[0]
    xmax = mm_ref[1]
    q = jnp.float32(_Q)
    t = (x - xmin) / xmax * q
    idx = jnp.clip(jnp.floor(t).astype(jnp.int32), 0, _Q - 1)
    a8 = idx * (_Q // 2)

    lane = jax.lax.broadcasted_iota(jnp.int32, (_C, _H, _W), 2)
    sub = jax.lax.broadcasted_iota(jnp.int32, (_C, _H, _W), 1)

    tab = jnp.broadcast_to(tab_ref[0][None, None, :], (_C, _H, _W))

    acc = jnp.zeros((_C, _H, _W), jnp.float32)
    for dw in (-1, 0, 1):
        xw = _shift_w(x, dw, lane, True)
        bw = _shift_w(idx, dw, lane, True)
        for dh in (-1, 0, 1):
            xwh = _shift_h(xw, dh, sub, True)
            bwh = _shift_h(bw, dh, sub, True)
            for dc in (-1, 0, 1):
                xs = _shift_c(xwh, dc)
                bs = _shift_c(bwh, dc)
                pair = a8 + (bs >> 1)
                u = jnp.take_along_axis(tab, pair, axis=2)
                odd = (bs & 1) == 1
                bits = jnp.where(odd, u & jnp.int32(-65536), u << 16)
                val = pltpu.bitcast(bits, jnp.float32)
                ft = f_ref[(dc + 1) * 9 + (dh + 1) * 3 + (dw + 1)]
                acc = acc + (ft * xs) * val
    o_ref[0] = acc


def _pack_co_table(co_matrix):
    cb = co_matrix.reshape(-1).astype(jnp.bfloat16)
    u16 = jax.lax.bitcast_convert_type(cb, jnp.uint16).astype(jnp.uint32)
    packed = u16[0::2] | (u16[1::2] << 16)
    return packed.astype(jnp.int32).reshape(1, 128)


def _local_minmax(x_s):
    xr = x_s.reshape(x_s.shape[0] * _C * _H, _W)
    return pl.pallas_call(
        _minmax_body,
        out_shape=jax.ShapeDtypeStruct((2,), jnp.float32),
        in_specs=[pl.BlockSpec(memory_space=pltpu.VMEM)],
        out_specs=pl.BlockSpec(memory_space=pltpu.SMEM),
    )(xr)


def _main_call(mm, x_s, co_matrix, spatial_filter):
    tab = _pack_co_table(co_matrix)
    f = spatial_filter.reshape(27)
    nloc = x_s.shape[0]
    return pl.pallas_call(
        _main_body,
        grid=(nloc,),
        out_shape=jax.ShapeDtypeStruct((nloc, _C, _H, _W), jnp.float32),
        in_specs=[
            pl.BlockSpec(memory_space=pltpu.SMEM),
            pl.BlockSpec(memory_space=pltpu.SMEM),
            pl.BlockSpec((1, 128), lambda n: (0, 0)),
            pl.BlockSpec((1, _C, _H, _W), lambda n: (n, 0, 0, 0)),
        ],
        out_specs=pl.BlockSpec((1, _C, _H, _W), lambda n: (n, 0, 0, 0)),
        compiler_params=pltpu.CompilerParams(
            dimension_semantics=("arbitrary",),
        ),
    )(mm, f, tab, x_s)


def kernel(x, co_matrix, spatial_filter):
    devs = jax.devices()
    if len(devs) < 2:
        mm = _local_minmax(x)
        return _main_call(mm, x, co_matrix, spatial_filter)

    mesh = Mesh(np.array(devs[:2]), ("d",))

    def shard_fn(x_full, co_s, f_s):
        mm = _local_minmax(x_full)
        i = jax.lax.axis_index("d")
        x_s = jax.lax.dynamic_slice_in_dim(x_full, i * (_N // 2), _N // 2, 0)
        return _main_call(mm, x_s, co_s, f_s)

    fn = jax.shard_map(
        shard_fn,
        mesh=mesh,
        in_specs=(P(), P(), P()),
        out_specs=P("d"),
        check_vma=False,
    )
    return fn(x, co_matrix, spatial_filter)
